# Initial kernel scaffold; baseline (speedup 1.0000x reference)
import jax
import jax.numpy as jnp
from jax.experimental import pallas as pl


def kernel(x, table):
    raise NotImplementedError("write your pallas kernel here")


# speedup vs baseline: 3.8571x; 3.8571x over previous
import functools

import jax
import jax.numpy as jnp
from jax import lax
from jax.experimental import pallas as pl
from jax.experimental.pallas import tpu as pltpu
from jax.experimental.pallas import tpu_sc as plsc

D = 64
IB = 128
CHUNK = 1024


def _make_gather(n_rows: int):
    info = plsc.get_sparse_core_info()
    nc, ns = info.num_cores, info.num_subcores
    nw = nc * ns
    per_w = n_rows // nw
    n_chunks = per_w // CHUNK
    n_sub = CHUNK // IB
    mesh = plsc.VectorSubcoreMesh(core_axis_name="c", subcore_axis_name="s")

    @functools.partial(
        pl.kernel,
        out_type=jax.ShapeDtypeStruct((n_rows, D), jnp.float32),
        mesh=mesh,
        compiler_params=pltpu.CompilerParams(
            use_tc_tiling_on_sc=False, needs_layout_passes=False),
        scratch_types=[
            pltpu.VMEM((n_sub, IB), jnp.int32),
            pltpu.VMEM((CHUNK, D), jnp.float32),
            pltpu.VMEM((CHUNK,), jnp.float32),
            pltpu.SemaphoreType.DMA,
        ],
    )
    def gather_kernel(tpad_hbm, idx_hbm, dur_hbm, out_hbm, idx_v, rows_v,
                      dur_v, sem):
        wid = lax.axis_index("s") * nc + lax.axis_index("c")
        base0 = wid * per_w

        def chunk_body(i, _):
            base = base0 + i * CHUNK
            irow = pl.multiple_of(base // IB, 8)
            pltpu.sync_copy(idx_hbm.at[pl.ds(irow, n_sub)], idx_v)
            pltpu.sync_copy(dur_hbm.at[pl.ds(base, CHUNK)], dur_v)
            copies = []
            for j in range(n_sub):
                copies.append(
                    pltpu.async_copy(
                        tpad_hbm.at[idx_v.at[j]],
                        rows_v.at[pl.ds(j * IB, IB)],
                        sem,
                    ))
            for c in copies:
                c.wait()
            col = jnp.full((16,), D - 1, jnp.int32)
            for k in range(CHUNK // 16):
                rows = lax.iota(jnp.int32, 16) + (k * 16)
                v = dur_v[pl.ds(k * 16, 16)]
                plsc.store_scatter(rows_v, [rows, col], v)
            pltpu.sync_copy(rows_v, out_hbm.at[pl.ds(base, CHUNK)])
            return 0

        lax.fori_loop(0, n_chunks, chunk_body, 0)

    return gather_kernel


def kernel(x, table):
    b, l, _ = x.shape
    n = b * l
    idx = x[..., 0].astype(jnp.int32).reshape(n // IB, IB)
    dur = x[..., 1].reshape(n)
    tpad = jnp.pad(table, ((0, 0), (0, 1)))
    out = _make_gather(n)(tpad, idx, dur)
    return out.reshape(b, l, D)


# speedup vs baseline: 4.0858x; 1.0593x over previous
import functools

import jax
import jax.numpy as jnp
from jax import lax
from jax.experimental import pallas as pl
from jax.experimental.pallas import tpu as pltpu
from jax.experimental.pallas import tpu_sc as plsc

D = 64
IB = 128
CHUNK = 512


def _make_gather(n_rows: int):
    info = plsc.get_sparse_core_info()
    nc, ns = info.num_cores, info.num_subcores
    nw = nc * ns
    per_w = n_rows // nw
    n_chunks = per_w // CHUNK
    n_sub = CHUNK // IB
    mesh = plsc.VectorSubcoreMesh(core_axis_name="c", subcore_axis_name="s")

    @functools.partial(
        pl.kernel,
        out_type=jax.ShapeDtypeStruct((n_rows, D), jnp.float32),
        mesh=mesh,
        compiler_params=pltpu.CompilerParams(
            use_tc_tiling_on_sc=False, needs_layout_passes=False),
        scratch_types=[
            pltpu.VMEM((per_w // IB, IB), jnp.int32),
            pltpu.VMEM((per_w,), jnp.float32),
            pltpu.VMEM((CHUNK, D), jnp.float32),
            pltpu.VMEM((CHUNK, D), jnp.float32),
            pltpu.SemaphoreType.DMA,
            pltpu.SemaphoreType.DMA,
            pltpu.SemaphoreType.DMA,
            pltpu.SemaphoreType.DMA,
        ],
    )
    def gather_kernel(tpad_hbm, idx_hbm, dur_hbm, out_hbm, idx_v, dur_v,
                      rows0, rows1, gsem0, gsem1, osem0, osem1):
        wid = lax.axis_index("s") * nc + lax.axis_index("c")
        base0 = pl.multiple_of(wid * per_w, CHUNK)
        rows = (rows0, rows1)
        gsem = (gsem0, gsem1)
        osem = (osem0, osem1)

        irow = pl.multiple_of(base0 // IB, 8)
        pltpu.sync_copy(idx_hbm.at[pl.ds(irow, per_w // IB)], idx_v)
        pltpu.sync_copy(dur_hbm.at[pl.ds(base0, per_w)], dur_v)

        def issue_gather(c, bi):
            for jj in range(n_sub):
                pltpu.async_copy(
                    tpad_hbm.at[idx_v.at[c * n_sub + jj]],
                    rows[bi].at[pl.ds(jj * IB, IB)],
                    gsem[bi],
                )

        def drain_gather(bi):
            pltpu.make_async_copy(
                out_hbm.at[pl.ds(0, CHUNK)], rows[bi], gsem[bi]).wait()

        def drain_out(c, bi):
            pltpu.make_async_copy(
                rows[bi], out_hbm.at[pl.ds(0, CHUNK)], osem[bi]).wait()

        def finish_chunk(c, bi):
            drain_gather(bi)
            col = jnp.full((16,), D - 1, jnp.int32)
            for k in range(CHUNK // 16):
                r = lax.iota(jnp.int32, 16) + (k * 16)
                v = dur_v[pl.ds(c * CHUNK + k * 16, 16)]
                plsc.store_scatter(rows[bi], [r, col], v)
            obase = pl.multiple_of(base0 + c * CHUNK, CHUNK)
            pltpu.async_copy(rows[bi], out_hbm.at[pl.ds(obase, CHUNK)],
                             osem[bi])

        issue_gather(0, 0)

        def pair_body(p, _):
            c0 = p * 2

            @pl.when(c0 + 1 < n_chunks)
            def _():
                pl.when(c0 >= 2)(lambda: drain_out(c0 + 1 - 2, 1))
                issue_gather(c0 + 1, 1)

            finish_chunk(c0, 0)

            @pl.when(c0 + 2 < n_chunks)
            def _():
                drain_out(c0 + 2 - 2, 0)
                issue_gather(c0 + 2, 0)

            pl.when(c0 + 1 < n_chunks)(lambda: finish_chunk(c0 + 1, 1))
            return 0

        lax.fori_loop(0, (n_chunks + 1) // 2, pair_body, 0)
        drain_out(n_chunks - 2, (n_chunks - 2) % 2)
        drain_out(n_chunks - 1, (n_chunks - 1) % 2)

    return gather_kernel


def kernel(x, table):
    b, l, _ = x.shape
    n = b * l
    idx = x[..., 0].astype(jnp.int32).reshape(n // IB, IB)
    dur = x[..., 1].reshape(n)
    tpad = jnp.pad(table, ((0, 0), (0, 1)))
    out = _make_gather(n)(tpad, idx, dur)
    return out.reshape(b, l, D)
